# baseline (speedup 1.0000x reference)
import functools

import numpy as np
import jax
import jax.numpy as jnp
from jax import lax
from jax.experimental import pallas as pl
from jax.experimental.pallas import tpu as pltpu
from jax.experimental.pallas import tpu_sc as plsc

MAX_CNT = 100000
N_NODES = 10000
N_EDGES = 160000
D = 256
NC = 2
NS = 16
NW = NC * NS
CHUNK = 64
NCH_C0 = 50
NCH_C1 = 24
NCH_PAIR = NCH_C0 + NCH_C1
NCH_MAX = max(NCH_C0, NCH_C1)
RPW_MAX = CHUNK * NCH_MAX
U_PAD = NS * NCH_PAIR * CHUNK

_consts_cache = None


def _rotl_np(x, r):
    return ((x << np.uint32(r)) | (x >> np.uint32(32 - r))).astype(np.uint32)


def _threefry2x32_np(k1, k2, x0, x1):
    rotations = ((13, 15, 26, 6), (17, 29, 16, 24))
    ks = [np.uint32(k1), np.uint32(k2), np.uint32(k1 ^ k2 ^ np.uint32(0x1BD11BDA))]
    x = [(x0 + ks[0]).astype(np.uint32), (x1 + ks[1]).astype(np.uint32)]
    for i in range(5):
        for r in rotations[i % 2]:
            x[0] = (x[0] + x[1]).astype(np.uint32)
            x[1] = _rotl_np(x[1], r)
            x[1] = x[0] ^ x[1]
        x[0] = (x[0] + ks[(i + 1) % 3]).astype(np.uint32)
        x[1] = (x[1] + ks[(i + 2) % 3] + np.uint32(i + 1)).astype(np.uint32)
    return x


def _fry_bits_np(k1, k2, size):
    x0 = np.zeros(size, np.uint32)
    x1 = np.arange(size, dtype=np.uint32)
    b1, b2 = _threefry2x32_np(k1, k2, x0, x1)
    return b1 ^ b2


def _np_randint_key42(size, span):
    b1, b2 = _threefry2x32_np(np.uint32(0), np.uint32(42),
                              np.zeros(2, np.uint32),
                              np.arange(2, dtype=np.uint32))
    y = _fry_bits_np(b1[0], b2[0], size)
    z = _fry_bits_np(b1[1], b2[1], size)
    span_u = np.uint32(span)
    mj = np.uint32(65536) % span_u
    with np.errstate(over="ignore"):
        mult = (mj * mj) % span_u
    val = ((y % span_u) * mult + z % span_u) % span_u
    return val.astype(np.int32)


def _get_consts():
    global _consts_cache
    if _consts_cache is None:
        idx = _np_randint_key42(MAX_CNT, N_EDGES)
        counts = np.bincount(idx, minlength=N_EDGES).astype(np.float32)
        uniq = np.nonzero(counts > 0)[0].astype(np.int32)
        nz = np.zeros(U_PAD, np.int32)
        nz[: uniq.size] = uniq
        cnt = np.zeros(U_PAD, np.float32)
        cnt[: uniq.size] = counts[uniq]
        _consts_cache = (nz, cnt)
    return _consts_cache


def _rsqrt16(x):
    i = plsc.bitcast(x, jnp.int32)
    y = plsc.bitcast(jnp.full((16,), 0x5F3759DF, jnp.int32) - (i >> 1), jnp.float32)
    for _ in range(3):
        y = y * (1.5 - 0.5 * x * y * y)
    return y


@functools.partial(
    pl.kernel,
    out_type=jax.ShapeDtypeStruct((NW, 16), jnp.float32),
    mesh=plsc.VectorSubcoreMesh(core_axis_name="c", subcore_axis_name="s"),
    compiler_params=pltpu.CompilerParams(use_tc_tiling_on_sc=True,
                                         needs_layout_passes=False),
    scratch_types=[
        pltpu.VMEM((RPW_MAX,), jnp.int32),
        pltpu.VMEM((RPW_MAX,), jnp.int32),
        pltpu.VMEM((RPW_MAX,), jnp.int32),
        pltpu.VMEM((RPW_MAX,), jnp.float32),
        pltpu.VMEM((CHUNK, D), jnp.float32),
        pltpu.VMEM((CHUNK, D // 2), jnp.int32),
        pltpu.VMEM((CHUNK, D // 2), jnp.int32),
        pltpu.VMEM((CHUNK, D), jnp.float32),
        pltpu.VMEM((CHUNK, D // 2), jnp.int32),
        pltpu.VMEM((CHUNK, D // 2), jnp.int32),
        pltpu.VMEM((16,), jnp.float32),
        pltpu.VMEM((272,), jnp.float32),
        pltpu.VMEM((272,), jnp.float32),
        pltpu.VMEM((272,), jnp.float32),
        pltpu.SemaphoreType.DMA,
        pltpu.SemaphoreType.DMA,
        pltpu.SemaphoreType.DMA,
    ],
)
def _loss_kernel(node_hbm, edge_hbm, src_hbm, dst_hbm, uid_hbm, cnt_hbm, out_hbm,
                 uid_f, sid_f, tid_f, cnt_f, er_a, sr_a, tr_a, er_b, sr_b, tr_b,
                 accv, pst_d, pst_a, pst_b, sem_id, sem_a, sem_b):
    c = lax.axis_index("c")
    s = lax.axis_index("s")
    w = s * NC + c
    base = (s * NCH_PAIR + c * NCH_C0) * CHUNK

    def issue(g, er, sr, tr, sem):
        sl = pl.ds(g * CHUNK, CHUNK)
        pltpu.async_copy(edge_hbm.at[uid_f.at[sl]], er, sem)
        pltpu.async_copy(node_hbm.at[sid_f.at[sl]], sr, sem)
        pltpu.async_copy(node_hbm.at[tid_f.at[sl]], tr, sem)

    def drain(g, er, sr, tr, sem):
        sl = pl.ds(g * CHUNK, CHUNK)
        pltpu.make_async_copy(edge_hbm.at[uid_f.at[sl]], er, sem).wait()
        pltpu.make_async_copy(node_hbm.at[sid_f.at[sl]], sr, sem).wait()
        pltpu.make_async_copy(node_hbm.at[tid_f.at[sl]], tr, sem).wait()

    lanes = lax.broadcasted_iota(jnp.int32, (16,), 0)
    zero16 = jnp.zeros((16,), jnp.float32)

    def tree(ps):
        while len(ps) > 1:
            ps = [ps[i] + ps[i + 1] for i in range(0, len(ps), 2)]
        return ps[0]

    lane17 = lanes * 17

    def compute(g, er, sr, tr, acc):
        def grp_body(k, acc_in):
            def row_body(i, carry):
                r = k * 16 + i
                pd = []
                pa = []
                pb = []
                for j in range(D // 32):
                    slw = pl.ds(j * 16, 16)
                    s_lo, s_hi = plsc.unpack(
                        plsc.bitcast(sr[r, slw], jnp.bfloat16),
                        format=plsc.PackFormat.INTERLEAVED)
                    t_lo, t_hi = plsc.unpack(
                        plsc.bitcast(tr[r, slw], jnp.bfloat16),
                        format=plsc.PackFormat.INTERLEAVED)
                    e_lo = er[r, pl.ds(j * 32, 16)]
                    e_hi = er[r, pl.ds(j * 32 + 16, 16)]
                    for a, t in ((s_lo + e_lo, t_lo), (s_hi + e_hi, t_hi)):
                        pd.append(a * t)
                        pa.append(a * a)
                        pb.append(t * t)
                pst_d[pl.ds(i * 17, 16)] = tree(pd)
                pst_a[pl.ds(i * 17, 16)] = tree(pa)
                pst_b[pl.ds(i * 17, 16)] = tree(pb)
                return carry

            lax.fori_loop(0, 16, row_body, 0, unroll=2)
            dvec = tree([plsc.load_gather(pst_d, [lane17 + c]) for c in range(16)])
            avec = tree([plsc.load_gather(pst_a, [lane17 + c]) for c in range(16)])
            bvec = tree([plsc.load_gather(pst_b, [lane17 + c]) for c in range(16)])
            x = jnp.maximum(avec, 1e-12) * jnp.maximum(bvec, 1e-12)
            sim = dvec * _rsqrt16(x)
            cv = cnt_f[pl.ds(g * CHUNK + k * 16, 16)]
            return acc_in + cv * (1.0 - sim)

        return lax.fori_loop(0, CHUNK // 16, grp_body, acc, unroll=False)

    def run(nch):
        rows = nch * CHUNK
        pltpu.sync_copy(uid_hbm.at[pl.ds(base, rows)], uid_f.at[pl.ds(0, rows)])
        pltpu.sync_copy(cnt_hbm.at[pl.ds(base, rows)], cnt_f.at[pl.ds(0, rows)])
        usl = uid_f.at[pl.ds(0, rows)]
        ssl = sid_f.at[pl.ds(0, rows)]
        tsl = tid_f.at[pl.ds(0, rows)]
        pltpu.async_copy(src_hbm.at[usl], ssl, sem_id)
        pltpu.async_copy(dst_hbm.at[usl], tsl, sem_id)
        pltpu.make_async_copy(src_hbm.at[usl], ssl, sem_id).wait()
        pltpu.make_async_copy(dst_hbm.at[usl], tsl, sem_id).wait()

        issue(0, er_a, sr_a, tr_a, sem_a)

        def body(gg, acc):
            g0 = 2 * gg
            g1 = g0 + 1
            issue(g1, er_b, sr_b, tr_b, sem_b)
            drain(g0, er_a, sr_a, tr_a, sem_a)
            acc = compute(g0, er_a, sr_a, tr_a, acc)

            @pl.when(g1 + 1 < nch)
            def _():
                issue(g1 + 1, er_a, sr_a, tr_a, sem_a)

            drain(g1, er_b, sr_b, tr_b, sem_b)
            return compute(g1, er_b, sr_b, tr_b, acc)

        acc = lax.fori_loop(0, nch // 2, body, jnp.zeros((16,), jnp.float32),
                            unroll=False)
        accv[...] = acc
        pltpu.sync_copy(accv, out_hbm.at[w])

    @pl.when(c == 0)
    def _():
        run(NCH_C0)

    @pl.when(c == 1)
    def _():
        run(NCH_C1)


def kernel(node_embed, edge_embed, node_scores, edge_idx, labels, mini_batch_id):
    nz, cnt = _get_consts()
    uid1 = jnp.asarray(nz)
    cnt1 = jnp.asarray(cnt)
    src = edge_idx[0].astype(jnp.int32)
    dst = edge_idx[1].astype(jnp.int32)
    u = jax.lax.bitcast_convert_type(node_embed, jnp.uint32)
    v = u.reshape(N_NODES, D // 32, 2, 16)
    words = (v[:, :, 1, :] & np.uint32(0xFFFF0000)) | (v[:, :, 0, :] >> 16)
    node_i32 = jax.lax.bitcast_convert_type(
        words.reshape(N_NODES, D // 2), jnp.int32)
    out = _loss_kernel(node_i32, edge_embed, src, dst, uid1, cnt1)
    return jnp.sum(out)


# speedup vs baseline: 1.0430x; 1.0430x over previous
import functools

import numpy as np
import jax
import jax.numpy as jnp
from jax import lax
from jax.experimental import pallas as pl
from jax.experimental.pallas import tpu as pltpu
from jax.experimental.pallas import tpu_sc as plsc

MAX_CNT = 100000
N_NODES = 10000
N_EDGES = 160000
D = 256
NC = 2
NS = 16
NW = NC * NS
CHUNK = 64
NCH_C0 = 48
NCH_C1 = 26
NCH_PAIR = NCH_C0 + NCH_C1
NCH_MAX = max(NCH_C0, NCH_C1)
RPW_MAX = CHUNK * NCH_MAX
U_PAD = NS * NCH_PAIR * CHUNK

_consts_cache = None


def _rotl_np(x, r):
    return ((x << np.uint32(r)) | (x >> np.uint32(32 - r))).astype(np.uint32)


def _threefry2x32_np(k1, k2, x0, x1):
    rotations = ((13, 15, 26, 6), (17, 29, 16, 24))
    ks = [np.uint32(k1), np.uint32(k2), np.uint32(k1 ^ k2 ^ np.uint32(0x1BD11BDA))]
    x = [(x0 + ks[0]).astype(np.uint32), (x1 + ks[1]).astype(np.uint32)]
    for i in range(5):
        for r in rotations[i % 2]:
            x[0] = (x[0] + x[1]).astype(np.uint32)
            x[1] = _rotl_np(x[1], r)
            x[1] = x[0] ^ x[1]
        x[0] = (x[0] + ks[(i + 1) % 3]).astype(np.uint32)
        x[1] = (x[1] + ks[(i + 2) % 3] + np.uint32(i + 1)).astype(np.uint32)
    return x


def _fry_bits_np(k1, k2, size):
    x0 = np.zeros(size, np.uint32)
    x1 = np.arange(size, dtype=np.uint32)
    b1, b2 = _threefry2x32_np(k1, k2, x0, x1)
    return b1 ^ b2


def _np_randint_key42(size, span):
    b1, b2 = _threefry2x32_np(np.uint32(0), np.uint32(42),
                              np.zeros(2, np.uint32),
                              np.arange(2, dtype=np.uint32))
    y = _fry_bits_np(b1[0], b2[0], size)
    z = _fry_bits_np(b1[1], b2[1], size)
    span_u = np.uint32(span)
    mj = np.uint32(65536) % span_u
    with np.errstate(over="ignore"):
        mult = (mj * mj) % span_u
    val = ((y % span_u) * mult + z % span_u) % span_u
    return val.astype(np.int32)


def _get_consts():
    global _consts_cache
    if _consts_cache is None:
        idx = _np_randint_key42(MAX_CNT, N_EDGES)
        counts = np.bincount(idx, minlength=N_EDGES).astype(np.float32)
        uniq = np.nonzero(counts > 0)[0].astype(np.int32)
        nz = np.zeros(U_PAD, np.int32)
        nz[: uniq.size] = uniq
        cnt = np.zeros(U_PAD, np.float32)
        cnt[: uniq.size] = counts[uniq]
        _consts_cache = (nz, cnt)
    return _consts_cache


def _rsqrt16(x):
    i = plsc.bitcast(x, jnp.int32)
    y = plsc.bitcast(jnp.full((16,), 0x5F3759DF, jnp.int32) - (i >> 1), jnp.float32)
    for _ in range(3):
        y = y * (1.5 - 0.5 * x * y * y)
    return y


@functools.partial(
    pl.kernel,
    out_type=jax.ShapeDtypeStruct((NW, 16), jnp.float32),
    mesh=plsc.VectorSubcoreMesh(core_axis_name="c", subcore_axis_name="s"),
    compiler_params=pltpu.CompilerParams(use_tc_tiling_on_sc=True,
                                         needs_layout_passes=False),
    scratch_types=[
        pltpu.VMEM((RPW_MAX,), jnp.int32),
        pltpu.VMEM((RPW_MAX,), jnp.int32),
        pltpu.VMEM((RPW_MAX,), jnp.int32),
        pltpu.VMEM((RPW_MAX,), jnp.float32),
        pltpu.VMEM((CHUNK, D), jnp.float32),
        pltpu.VMEM((CHUNK, D // 2), jnp.int32),
        pltpu.VMEM((CHUNK, D // 2), jnp.int32),
        pltpu.VMEM((CHUNK, D), jnp.float32),
        pltpu.VMEM((CHUNK, D // 2), jnp.int32),
        pltpu.VMEM((CHUNK, D // 2), jnp.int32),
        pltpu.VMEM((16,), jnp.float32),
        pltpu.VMEM((272,), jnp.float32),
        pltpu.VMEM((272,), jnp.float32),
        pltpu.VMEM((272,), jnp.float32),
        pltpu.SemaphoreType.DMA,
        pltpu.SemaphoreType.DMA,
        pltpu.SemaphoreType.DMA,
    ],
)
def _loss_kernel(node_hbm, edge_hbm, src_hbm, dst_hbm, uid_hbm, cnt_hbm, out_hbm,
                 uid_f, sid_f, tid_f, cnt_f, er_a, sr_a, tr_a, er_b, sr_b, tr_b,
                 accv, pst_d, pst_a, pst_b, sem_id, sem_a, sem_b):
    c = lax.axis_index("c")
    s = lax.axis_index("s")
    w = s * NC + c
    base = (s * NCH_PAIR + c * NCH_C0) * CHUNK

    def issue(g, er, sr, tr, sem):
        sl = pl.ds(g * CHUNK, CHUNK)
        pltpu.async_copy(edge_hbm.at[uid_f.at[sl]], er, sem)
        pltpu.async_copy(node_hbm.at[sid_f.at[sl]], sr, sem)
        pltpu.async_copy(node_hbm.at[tid_f.at[sl]], tr, sem)

    def drain(g, er, sr, tr, sem):
        sl = pl.ds(g * CHUNK, CHUNK)
        pltpu.make_async_copy(edge_hbm.at[uid_f.at[sl]], er, sem).wait()
        pltpu.make_async_copy(node_hbm.at[sid_f.at[sl]], sr, sem).wait()
        pltpu.make_async_copy(node_hbm.at[tid_f.at[sl]], tr, sem).wait()

    lanes = lax.broadcasted_iota(jnp.int32, (16,), 0)
    zero16 = jnp.zeros((16,), jnp.float32)

    def tree(ps):
        while len(ps) > 1:
            ps = [ps[i] + ps[i + 1] for i in range(0, len(ps), 2)]
        return ps[0]

    lane17 = lanes * 17

    def compute(g, er, sr, tr, acc):
        def grp_body(k, acc_in):
            def row_body(i, carry):
                r = k * 16 + i
                pd = []
                pa = []
                pb = []
                for j in range(D // 32):
                    slw = pl.ds(j * 16, 16)
                    s_lo, s_hi = plsc.unpack(
                        plsc.bitcast(sr[r, slw], jnp.bfloat16),
                        format=plsc.PackFormat.INTERLEAVED)
                    t_lo, t_hi = plsc.unpack(
                        plsc.bitcast(tr[r, slw], jnp.bfloat16),
                        format=plsc.PackFormat.INTERLEAVED)
                    e_lo = er[r, pl.ds(j * 32, 16)]
                    e_hi = er[r, pl.ds(j * 32 + 16, 16)]
                    for a, t in ((s_lo + e_lo, t_lo), (s_hi + e_hi, t_hi)):
                        pd.append(a * t)
                        pa.append(a * a)
                        pb.append(t * t)
                pst_d[pl.ds(i * 17, 16)] = tree(pd)
                pst_a[pl.ds(i * 17, 16)] = tree(pa)
                pst_b[pl.ds(i * 17, 16)] = tree(pb)
                return carry

            lax.fori_loop(0, 16, row_body, 0, unroll=False)
            dvec = tree([plsc.load_gather(pst_d, [lane17 + c]) for c in range(16)])
            avec = tree([plsc.load_gather(pst_a, [lane17 + c]) for c in range(16)])
            bvec = tree([plsc.load_gather(pst_b, [lane17 + c]) for c in range(16)])
            x = jnp.maximum(avec, 1e-12) * jnp.maximum(bvec, 1e-12)
            sim = dvec * _rsqrt16(x)
            cv = cnt_f[pl.ds(g * CHUNK + k * 16, 16)]
            return acc_in + cv * (1.0 - sim)

        return lax.fori_loop(0, CHUNK // 16, grp_body, acc, unroll=False)

    def run(nch):
        rows = nch * CHUNK
        pltpu.sync_copy(uid_hbm.at[pl.ds(base, rows)], uid_f.at[pl.ds(0, rows)])
        pltpu.sync_copy(cnt_hbm.at[pl.ds(base, rows)], cnt_f.at[pl.ds(0, rows)])
        usl = uid_f.at[pl.ds(0, rows)]
        ssl = sid_f.at[pl.ds(0, rows)]
        tsl = tid_f.at[pl.ds(0, rows)]
        pltpu.async_copy(src_hbm.at[usl], ssl, sem_id)
        pltpu.async_copy(dst_hbm.at[usl], tsl, sem_id)
        pltpu.make_async_copy(src_hbm.at[usl], ssl, sem_id).wait()
        pltpu.make_async_copy(dst_hbm.at[usl], tsl, sem_id).wait()

        issue(0, er_a, sr_a, tr_a, sem_a)

        def body(gg, acc):
            g0 = 2 * gg
            g1 = g0 + 1
            issue(g1, er_b, sr_b, tr_b, sem_b)
            drain(g0, er_a, sr_a, tr_a, sem_a)
            acc = compute(g0, er_a, sr_a, tr_a, acc)

            @pl.when(g1 + 1 < nch)
            def _():
                issue(g1 + 1, er_a, sr_a, tr_a, sem_a)

            drain(g1, er_b, sr_b, tr_b, sem_b)
            return compute(g1, er_b, sr_b, tr_b, acc)

        acc = lax.fori_loop(0, nch // 2, body, jnp.zeros((16,), jnp.float32),
                            unroll=False)
        accv[...] = acc
        pltpu.sync_copy(accv, out_hbm.at[w])

    @pl.when(c == 0)
    def _():
        run(NCH_C0)

    @pl.when(c == 1)
    def _():
        run(NCH_C1)


def kernel(node_embed, edge_embed, node_scores, edge_idx, labels, mini_batch_id):
    nz, cnt = _get_consts()
    uid1 = jnp.asarray(nz)
    cnt1 = jnp.asarray(cnt)
    src = edge_idx[0].astype(jnp.int32)
    dst = edge_idx[1].astype(jnp.int32)
    u = jax.lax.bitcast_convert_type(node_embed, jnp.uint32)
    v = u.reshape(N_NODES, D // 32, 2, 16)
    words = (v[:, :, 1, :] & np.uint32(0xFFFF0000)) | (v[:, :, 0, :] >> 16)
    node_i32 = jax.lax.bitcast_convert_type(
        words.reshape(N_NODES, D // 2), jnp.int32)
    out = _loss_kernel(node_i32, edge_embed, src, dst, uid1, cnt1)
    return jnp.sum(out)
